# baseline (speedup 1.0000x reference)
import functools

import jax
import jax.numpy as jnp
from jax import lax
from jax.experimental import pallas as pl
from jax.experimental.pallas import tpu as pltpu
from jax.experimental.pallas import tpu_sc as plsc

_VOCAB = 1000000
_DIM = 64
_BATCH = 16384
_LANES = 16

_NC = 2
_NS = 16
_NW = _NC * _NS
_BPW = _BATCH // _NW
_CHUNK = 32
_NCHUNK = _BPW // _CHUNK
_IROWS = _BPW // 128
_ROWS = _BATCH // 128
_NBUF = 2


def _lane_permute(x, idx):
    return lax.gather(
        x, idx[:, None],
        lax.GatherDimensionNumbers(
            offset_dims=(), collapsed_slice_dims=(0,), start_index_map=(0,)),
        slice_sizes=(1,),
        mode=lax.GatherScatterMode.PROMISE_IN_BOUNDS)


def _sc_body(t_hbm, c_hbm, tt_hbm, ct_hbm, out_hbm,
             tidx_v, cidx_v, trows, crows, scores, sem):
    wid = lax.axis_index("s") * _NC + lax.axis_index("c")
    base = wid * _IROWS

    pltpu.sync_copy(t_hbm.at[pl.ds(base, _IROWS)], tidx_v)
    pltpu.sync_copy(c_hbm.at[pl.ds(base, _IROWS)], cidx_v)

    def chunk_scalars(ci):
        r, c0 = ci >> 2, (ci & (128 // _CHUNK - 1)) * _CHUNK
        tis, cis = [], []
        for v in range(_CHUNK // _LANES):
            tvec = tidx_v[r, pl.ds(c0 + v * _LANES, _LANES)]
            cvec = cidx_v[r, pl.ds(c0 + v * _LANES, _LANES)]
            tis += [tvec[l] for l in range(_LANES)]
            cis += [cvec[l] for l in range(_LANES)]
        return tis, cis

    def row_copies(ti, ci_, slot, j):
        t_cp = (tt_hbm.at[pl.ds(ti, 1)], trows.at[slot, pl.ds(j, 1)])
        c_cp = (ct_hbm.at[pl.ds(ci_, 1)], crows.at[slot, pl.ds(j, 1)])
        return t_cp, c_cp

    def fire(ci, slot):
        tis, cis = chunk_scalars(ci)
        for j in range(_CHUNK):
            t_cp, c_cp = row_copies(tis[j], cis[j], slot, j)
            pltpu.async_copy(*t_cp, sem)
            pltpu.async_copy(*c_cp, sem)

    def drain(ci, slot):
        tis, cis = chunk_scalars(ci)
        for j in range(_CHUNK):
            t_cp, c_cp = row_copies(tis[j], cis[j], slot, j)
            pltpu.make_async_copy(*t_cp, sem).wait()
            pltpu.make_async_copy(*c_cp, sem).wait()

    lane = lax.iota(jnp.int32, _LANES)
    stages = [(lane ^ h, (lane & h) == 0) for h in (8, 4, 2, 1)]
    bitrev = (((lane & 1) << 3) | ((lane & 2) << 1)
              | ((lane & 4) >> 1) | ((lane & 8) >> 3))

    def merge(a, b, perm_h, mask_h):
        u = a + _lane_permute(a, perm_h)
        v = b + _lane_permute(b, perm_h)
        return jnp.where(mask_h, u, v)

    def tree(vecs):
        for perm_h, mask_h in stages:
            vecs = [merge(vecs[i], vecs[i + 1], perm_h, mask_h)
                    for i in range(0, len(vecs), 2)]
        return _lane_permute(vecs[0], bitrev)

    fire(0, 0)

    def chunk_body(ci, _):
        slot = lax.rem(ci, _NBUF)
        drain(ci, slot)

        @pl.when(ci + 1 < _NCHUNK)
        def _():
            fire(ci + 1, lax.rem(ci + 1, _NBUF))

        for g in range(_CHUNK // _LANES):
            vecs = []
            for r in range(_LANES):
                j = g * _LANES + r
                acc = (trows[slot, j, pl.ds(0, _LANES)]
                       * crows[slot, j, pl.ds(0, _LANES)])
                for k in range(1, _DIM // _LANES):
                    acc = acc + (trows[slot, j, pl.ds(k * _LANES, _LANES)]
                                 * crows[slot, j, pl.ds(k * _LANES, _LANES)])
                vecs.append(acc)
            totals = tree(vecs)
            flat = ci * _CHUNK + g * _LANES
            scores[flat >> 7, pl.ds(flat & 127, _LANES)] = totals
        return 0

    lax.fori_loop(0, _NCHUNK, chunk_body, 0)

    pltpu.sync_copy(scores, out_hbm.at[pl.ds(base, _IROWS)])


@jax.jit
def _sc_scores(t_idx, c_idx, target_table, context_table):
    mesh = plsc.VectorSubcoreMesh(core_axis_name="c", subcore_axis_name="s")
    k = functools.partial(
        pl.kernel,
        mesh=mesh,
        out_type=jax.ShapeDtypeStruct((_ROWS, 128), jnp.float32),
        scratch_types=[
            pltpu.VMEM((_IROWS, 128), jnp.int32),
            pltpu.VMEM((_IROWS, 128), jnp.int32),
            pltpu.VMEM((_NBUF, _CHUNK, _DIM), jnp.float32),
            pltpu.VMEM((_NBUF, _CHUNK, _DIM), jnp.float32),
            pltpu.VMEM((_IROWS, 128), jnp.float32),
            pltpu.SemaphoreType.DMA,
        ],
    )(_sc_body)
    return k(t_idx, c_idx, target_table, context_table)


def kernel(target, context, target_table, context_table):
    t_idx = target.astype(jnp.int32).reshape(_ROWS, 128)
    c_idx = context.astype(jnp.int32).reshape(_ROWS, 128)
    out = _sc_scores(t_idx, c_idx, target_table, context_table)
    return out.reshape(_BATCH)


# speedup vs baseline: 1.0080x; 1.0080x over previous
import functools

import jax
import jax.numpy as jnp
from jax import lax
from jax.experimental import pallas as pl
from jax.experimental.pallas import tpu as pltpu
from jax.experimental.pallas import tpu_sc as plsc

_VOCAB = 1000000
_DIM = 64
_BATCH = 16384
_LANES = 16

_NC = 2
_NS = 16
_NW = _NC * _NS
_BPW = _BATCH // _NW
_CHUNK = 32
_NCHUNK = _BPW // _CHUNK
_IROWS = _BPW // 128
_ROWS = _BATCH // 128
_NBUF = 8


def _lane_permute(x, idx):
    return lax.gather(
        x, idx[:, None],
        lax.GatherDimensionNumbers(
            offset_dims=(), collapsed_slice_dims=(0,), start_index_map=(0,)),
        slice_sizes=(1,),
        mode=lax.GatherScatterMode.PROMISE_IN_BOUNDS)


def _sc_body(t_hbm, c_hbm, tt_hbm, ct_hbm, out_hbm,
             tidx_v, cidx_v, trows, crows, scores, sem):
    wid = lax.axis_index("s") * _NC + lax.axis_index("c")
    base = wid * _IROWS

    pltpu.sync_copy(t_hbm.at[pl.ds(base, _IROWS)], tidx_v)
    pltpu.sync_copy(c_hbm.at[pl.ds(base, _IROWS)], cidx_v)

    def chunk_scalars(ci):
        r, c0 = ci >> 2, (ci & (128 // _CHUNK - 1)) * _CHUNK
        tis, cis = [], []
        for v in range(_CHUNK // _LANES):
            tvec = tidx_v[r, pl.ds(c0 + v * _LANES, _LANES)]
            cvec = cidx_v[r, pl.ds(c0 + v * _LANES, _LANES)]
            tis += [tvec[l] for l in range(_LANES)]
            cis += [cvec[l] for l in range(_LANES)]
        return tis, cis

    def row_copies(ti, ci_, slot, j):
        t_cp = (tt_hbm.at[pl.ds(ti, 1)], trows.at[slot, pl.ds(j, 1)])
        c_cp = (ct_hbm.at[pl.ds(ci_, 1)], crows.at[slot, pl.ds(j, 1)])
        return t_cp, c_cp

    def fire(ci, slot):
        tis, cis = chunk_scalars(ci)
        for j in range(_CHUNK):
            t_cp, c_cp = row_copies(tis[j], cis[j], slot, j)
            pltpu.async_copy(*t_cp, sem)
            pltpu.async_copy(*c_cp, sem)

    def drain(ci, slot):
        tis, cis = chunk_scalars(ci)
        for j in range(_CHUNK):
            t_cp, c_cp = row_copies(tis[j], cis[j], slot, j)
            pltpu.make_async_copy(*t_cp, sem).wait()
            pltpu.make_async_copy(*c_cp, sem).wait()

    lane = lax.iota(jnp.int32, _LANES)
    stages = [(lane ^ h, (lane & h) == 0) for h in (8, 4, 2, 1)]
    bitrev = (((lane & 1) << 3) | ((lane & 2) << 1)
              | ((lane & 4) >> 1) | ((lane & 8) >> 3))

    def merge(a, b, perm_h, mask_h):
        u = a + _lane_permute(a, perm_h)
        v = b + _lane_permute(b, perm_h)
        return jnp.where(mask_h, u, v)

    def tree(vecs):
        for perm_h, mask_h in stages:
            vecs = [merge(vecs[i], vecs[i + 1], perm_h, mask_h)
                    for i in range(0, len(vecs), 2)]
        return _lane_permute(vecs[0], bitrev)

    for ci in range(_NBUF - 1):
        fire(ci, ci)

    def chunk_body(ci, _):
        slot = lax.rem(ci, _NBUF)
        drain(ci, slot)

        @pl.when(ci + _NBUF - 1 < _NCHUNK)
        def _():
            nxt = ci + _NBUF - 1
            fire(nxt, lax.rem(nxt, _NBUF))

        for g in range(_CHUNK // _LANES):
            vecs = []
            for r in range(_LANES):
                j = g * _LANES + r
                acc = (trows[slot, j, pl.ds(0, _LANES)]
                       * crows[slot, j, pl.ds(0, _LANES)])
                for k in range(1, _DIM // _LANES):
                    acc = acc + (trows[slot, j, pl.ds(k * _LANES, _LANES)]
                                 * crows[slot, j, pl.ds(k * _LANES, _LANES)])
                vecs.append(acc)
            totals = tree(vecs)
            flat = ci * _CHUNK + g * _LANES
            scores[flat >> 7, pl.ds(flat & 127, _LANES)] = totals
        return 0

    lax.fori_loop(0, _NCHUNK, chunk_body, 0)

    pltpu.sync_copy(scores, out_hbm.at[pl.ds(base, _IROWS)])


@jax.jit
def _sc_scores(t_idx, c_idx, target_table, context_table):
    mesh = plsc.VectorSubcoreMesh(core_axis_name="c", subcore_axis_name="s")
    k = functools.partial(
        pl.kernel,
        mesh=mesh,
        out_type=jax.ShapeDtypeStruct((_ROWS, 128), jnp.float32),
        scratch_types=[
            pltpu.VMEM((_IROWS, 128), jnp.int32),
            pltpu.VMEM((_IROWS, 128), jnp.int32),
            pltpu.VMEM((_NBUF, _CHUNK, _DIM), jnp.float32),
            pltpu.VMEM((_NBUF, _CHUNK, _DIM), jnp.float32),
            pltpu.VMEM((_IROWS, 128), jnp.float32),
            pltpu.SemaphoreType.DMA,
        ],
    )(_sc_body)
    return k(t_idx, c_idx, target_table, context_table)


def kernel(target, context, target_table, context_table):
    t_idx = target.astype(jnp.int32).reshape(_ROWS, 128)
    c_idx = context.astype(jnp.int32).reshape(_ROWS, 128)
    out = _sc_scores(t_idx, c_idx, target_table, context_table)
    return out.reshape(_BATCH)
